# Initial kernel scaffold; baseline (speedup 1.0000x reference)
import jax
import jax.numpy as jnp
from jax.experimental import pallas as pl


def kernel(x, edge_index, W1, b1, W_mu, b_mu, W_logvar, b_logvar):
    raise NotImplementedError("write your pallas kernel here")


# speedup vs baseline: 16.5334x; 16.5334x over previous
import functools

import jax
import jax.numpy as jnp
from jax import lax
from jax.experimental import pallas as pl
from jax.experimental.pallas import tpu as pltpu
from jax.experimental.pallas import tpu_sc as plsc

N_NODES = 10000
N_PAD = 10240
N_EDGES = 320000
D = 128
LATENT = 64

NC = 2
NS = 16
NW = NC * NS
L = 16
EPW = N_EDGES // NW
B = 80
NBLK = EPW // B
DROWS = N_PAD // D
RPT = N_PAD // NS

_mesh = plsc.VectorSubcoreMesh(
    core_axis_name="c", subcore_axis_name="s", num_cores=NC, num_subcores=NS
)
_sc_params = pltpu.CompilerParams(needs_layout_passes=False)


def _zero_rows(ref, nrows):
  zero16 = jnp.zeros((L,), jnp.float32)

  def body(r, _):
    for c in range(D // L):
      ref[r, pl.ds(c * L, L)] = zero16
    return 0

  lax.fori_loop(0, nrows, body, 0)


_NPT = N_PAD // NS


def _deg_body(dst_hbm, out0_hbm, out1_hbm, dst_stage, deg_local, red, acc,
              slots):
  cid = lax.axis_index("c")
  sid = lax.axis_index("s")
  wid = sid * NC + cid

  zero16 = jnp.zeros((L,), jnp.float32)

  def zloc(i, _):
    deg_local[pl.ds(i * L, L)] = zero16
    return 0

  lax.fori_loop(0, N_PAD // L, zloc, 0)

  pltpu.sync_copy(dst_hbm.at[pl.ds(wid * EPW, EPW)], dst_stage)
  ones16 = jnp.ones((L,), jnp.float32)

  def hist(i, _):
    d = dst_stage[pl.ds(i * L, L)]
    plsc.addupdate_scatter(deg_local, [d], ones16)
    return 0

  lax.fori_loop(0, EPW // L, hist, 0)

  pltpu.sync_copy(deg_local, slots.at[sid])
  plsc.subcore_barrier()
  for k in range(NS):
    pltpu.sync_copy(slots.at[k, pl.ds(sid * _NPT, _NPT)], red.at[k])

  def rbody(j, _):
    s = red[0, pl.ds(j * L, L)]
    for k in range(1, NS):
      s = s + red[k, pl.ds(j * L, L)]
    acc[pl.ds(j * L, L)] = s
    return 0

  lax.fori_loop(0, _NPT // L, rbody, 0)

  @pl.when(cid == 0)
  def _():
    pltpu.sync_copy(acc, out0_hbm.at[pl.ds(sid * _NPT, _NPT)])

  @pl.when(cid == 1)
  def _():
    pltpu.sync_copy(acc, out1_hbm.at[pl.ds(sid * _NPT, _NPT)])


_deg_call = pl.kernel(
    _deg_body,
    out_type=(jax.ShapeDtypeStruct((N_PAD,), jnp.float32),
              jax.ShapeDtypeStruct((N_PAD,), jnp.float32)),
    mesh=_mesh,
    scratch_types=[
        pltpu.VMEM((EPW,), jnp.int32),
        pltpu.VMEM((N_PAD,), jnp.float32),
        pltpu.VMEM((NS, _NPT), jnp.float32),
        pltpu.VMEM((_NPT,), jnp.float32),
        pltpu.VMEM_SHARED((NS, N_PAD), jnp.float32),
    ],
    compiler_params=_sc_params,
)


def _spmm_body(y_hbm, src_hbm, dst_hbm, out_hbm, src_idx, dst_idx, rows, zbuf,
               sem, accum):
  cid = lax.axis_index("c")
  sid = lax.axis_index("s")
  wid = sid * NC + cid

  _zero_rows(zbuf, 128)
  for k in range(RPT // 128):
    pltpu.sync_copy(zbuf, accum.at[pl.ds(sid * RPT + k * 128, 128)])
  plsc.subcore_barrier()

  ebase = wid * EPW

  def blk(j, _):
    base = ebase + j * B
    pltpu.sync_copy(src_hbm.at[pl.ds(base, B)], src_idx)
    pltpu.sync_copy(dst_hbm.at[pl.ds(base, B)], dst_idx)
    pltpu.async_copy(y_hbm.at[src_idx], rows, sem).wait()
    pltpu.sync_copy(rows, accum.at[dst_idx], add=True)
    return 0

  lax.fori_loop(0, NBLK, blk, 0)

  plsc.subcore_barrier()
  pltpu.sync_copy(accum.at[pl.ds(sid * RPT, RPT)],
                  out_hbm.at[cid, pl.ds(sid * RPT, RPT)])


_spmm_call = pl.kernel(
    _spmm_body,
    out_type=jax.ShapeDtypeStruct((NC, N_PAD, D), jnp.float32),
    mesh=_mesh,
    scratch_types=[
        pltpu.VMEM((B,), jnp.int32),
        pltpu.VMEM((B,), jnp.int32),
        pltpu.VMEM((B, D), jnp.float32),
        pltpu.VMEM((128, D), jnp.float32),
        pltpu.SemaphoreType.DMA,
        pltpu.VMEM_SHARED((N_PAD, D), jnp.float32),
    ],
    compiler_params=_sc_params,
)


_RB = 1024
_GRID = N_PAD // _RB


def _lin1_body(x_ref, w_ref, d0_ref, d1_ref, y_ref, dinv_ref):
  dinv = lax.rsqrt(d0_ref[...] + d1_ref[...] + 1.0)
  h = jnp.dot(x_ref[...], w_ref[...], preferred_element_type=jnp.float32)
  y_ref[...] = dinv * h
  dinv_ref[...] = dinv


_lin1 = pl.pallas_call(
    _lin1_body,
    grid=(_GRID,),
    in_specs=[
        pl.BlockSpec((_RB, D), lambda i: (i, 0)),
        pl.BlockSpec((D, D), lambda i: (0, 0)),
        pl.BlockSpec((_RB, 1), lambda i: (i, 0)),
        pl.BlockSpec((_RB, 1), lambda i: (i, 0)),
    ],
    out_specs=[
        pl.BlockSpec((_RB, D), lambda i: (i, 0)),
        pl.BlockSpec((_RB, 1), lambda i: (i, 0)),
    ],
    out_shape=[
        jax.ShapeDtypeStruct((N_PAD, D), jnp.float32),
        jax.ShapeDtypeStruct((N_PAD, 1), jnp.float32),
    ],
)


def _lin2_body(z0_ref, z1_ref, y1_ref, dinv_ref, b1_ref, wc_ref, y2_ref):
  dinv = dinv_ref[...]
  h = dinv * (z0_ref[...] + z1_ref[...] + y1_ref[...]) + b1_ref[...]
  h = jnp.maximum(h, 0.0)
  y2_ref[...] = dinv * jnp.dot(h, wc_ref[...],
                               preferred_element_type=jnp.float32)


_lin2 = pl.pallas_call(
    _lin2_body,
    grid=(_GRID,),
    in_specs=[
        pl.BlockSpec((_RB, D), lambda i: (i, 0)),
        pl.BlockSpec((_RB, D), lambda i: (i, 0)),
        pl.BlockSpec((_RB, D), lambda i: (i, 0)),
        pl.BlockSpec((_RB, 1), lambda i: (i, 0)),
        pl.BlockSpec((1, D), lambda i: (0, 0)),
        pl.BlockSpec((D, D), lambda i: (0, 0)),
    ],
    out_specs=pl.BlockSpec((_RB, D), lambda i: (i, 0)),
    out_shape=jax.ShapeDtypeStruct((N_PAD, D), jnp.float32),
)


def _lin3_body(z0_ref, z1_ref, y2_ref, dinv_ref, bc_ref, o_ref):
  o_ref[...] = dinv_ref[...] * (z0_ref[...] + z1_ref[...] + y2_ref[...]) \
      + bc_ref[...]


_lin3 = pl.pallas_call(
    _lin3_body,
    grid=(_GRID,),
    in_specs=[
        pl.BlockSpec((_RB, D), lambda i: (i, 0)),
        pl.BlockSpec((_RB, D), lambda i: (i, 0)),
        pl.BlockSpec((_RB, D), lambda i: (i, 0)),
        pl.BlockSpec((_RB, 1), lambda i: (i, 0)),
        pl.BlockSpec((1, D), lambda i: (0, 0)),
    ],
    out_specs=pl.BlockSpec((_RB, D), lambda i: (i, 0)),
    out_shape=jax.ShapeDtypeStruct((N_PAD, D), jnp.float32),
)


def kernel(x, edge_index, W1, b1, W_mu, b_mu, W_logvar, b_logvar):
  xp = jnp.pad(x, ((0, N_PAD - N_NODES), (0, 0)))
  Wc = jnp.concatenate([W_mu, W_logvar], axis=1)
  bc = jnp.concatenate([b_mu, b_logvar]).reshape(1, D)

  src = edge_index[0]
  dst = edge_index[1]

  dg0, dg1 = _deg_call(dst)
  d0 = dg0.reshape(N_PAD, 1)
  d1 = dg1.reshape(N_PAD, 1)

  y1, dinv = _lin1(xp, W1, d0, d1)
  z1 = _spmm_call(y1, src, dst)
  y2 = _lin2(z1[0], z1[1], y1, dinv, b1.reshape(1, D), Wc)
  z2 = _spmm_call(y2, src, dst)
  o = _lin3(z2[0], z2[1], y2, dinv, bc)
  return o[:N_NODES, :LATENT], o[:N_NODES, LATENT:]


# speedup vs baseline: 34.2850x; 2.0737x over previous
import functools

import jax
import jax.numpy as jnp
from jax import lax
from jax.experimental import pallas as pl
from jax.experimental.pallas import tpu as pltpu
from jax.experimental.pallas import tpu_sc as plsc

N_NODES = 10000
N_PAD = 10240
N_EDGES = 320000
D = 128
LATENT = 64

NC = 2
NS = 16
NW = NC * NS
L = 16
EPW = N_EDGES // NW
B = 80
NBLK = EPW // B
DROWS = N_PAD // D
RPT = N_PAD // NS

_mesh = plsc.VectorSubcoreMesh(
    core_axis_name="c", subcore_axis_name="s", num_cores=NC, num_subcores=NS
)
_sc_params = pltpu.CompilerParams(needs_layout_passes=False)


def _zero_rows(ref, nrows):
  zero16 = jnp.zeros((L,), jnp.float32)

  def body(r, _):
    for c in range(D // L):
      ref[r, pl.ds(c * L, L)] = zero16
    return 0

  lax.fori_loop(0, nrows, body, 0)


_NPT = N_PAD // NS


def _deg_body(dst_hbm, out0_hbm, out1_hbm, dst_stage, deg_local, red, acc,
              slots):
  cid = lax.axis_index("c")
  sid = lax.axis_index("s")
  wid = sid * NC + cid

  zero16 = jnp.zeros((L,), jnp.float32)

  def zloc(i, _):
    deg_local[pl.ds(i * L, L)] = zero16
    return 0

  lax.fori_loop(0, N_PAD // L, zloc, 0)

  pltpu.sync_copy(dst_hbm.at[pl.ds(wid * EPW, EPW)], dst_stage)
  ones16 = jnp.ones((L,), jnp.float32)

  def hist(i, _):
    d = dst_stage[pl.ds(i * L, L)]
    plsc.addupdate_scatter(deg_local, [d], ones16)
    return 0

  lax.fori_loop(0, EPW // L, hist, 0)

  pltpu.sync_copy(deg_local, slots.at[sid])
  plsc.subcore_barrier()
  for k in range(NS):
    pltpu.sync_copy(slots.at[k, pl.ds(sid * _NPT, _NPT)], red.at[k])

  def rbody(j, _):
    s = red[0, pl.ds(j * L, L)]
    for k in range(1, NS):
      s = s + red[k, pl.ds(j * L, L)]
    acc[pl.ds(j * L, L)] = s
    return 0

  lax.fori_loop(0, _NPT // L, rbody, 0)

  @pl.when(cid == 0)
  def _():
    pltpu.sync_copy(acc, out0_hbm.at[pl.ds(sid * _NPT, _NPT)])

  @pl.when(cid == 1)
  def _():
    pltpu.sync_copy(acc, out1_hbm.at[pl.ds(sid * _NPT, _NPT)])


_deg_call = pl.kernel(
    _deg_body,
    out_type=(jax.ShapeDtypeStruct((N_PAD,), jnp.float32),
              jax.ShapeDtypeStruct((N_PAD,), jnp.float32)),
    mesh=_mesh,
    scratch_types=[
        pltpu.VMEM((EPW,), jnp.int32),
        pltpu.VMEM((N_PAD,), jnp.float32),
        pltpu.VMEM((NS, _NPT), jnp.float32),
        pltpu.VMEM((_NPT,), jnp.float32),
        pltpu.VMEM_SHARED((NS, N_PAD), jnp.float32),
    ],
    compiler_params=_sc_params,
)


def _spmm_body(y_hbm, src_hbm, dst_hbm, out_hbm, src_all, dst_all, rows0,
               rows1, zbuf, sem0, sem1, accum):
  cid = lax.axis_index("c")
  sid = lax.axis_index("s")
  wid = sid * NC + cid

  _zero_rows(zbuf, 32)
  for k in range(RPT // 32):
    pltpu.sync_copy(zbuf, accum.at[pl.ds(sid * RPT + k * 32, 32)])

  ebase = wid * EPW
  pltpu.sync_copy(src_hbm.at[pl.ds(ebase, EPW)], src_all)
  pltpu.sync_copy(dst_hbm.at[pl.ds(ebase, EPW)], dst_all)
  plsc.subcore_barrier()

  def gather(j, rows, sem):
    return pltpu.make_async_copy(
        y_hbm.at[src_all.at[pl.ds(j * B, B)]], rows, sem)

  def scatter(j, rows):
    pltpu.sync_copy(rows, accum.at[dst_all.at[pl.ds(j * B, B)]], add=True)

  gather(0, rows0, sem0).start()

  def pair(i, _):
    j = i * 2
    gather(j + 1, rows1, sem1).start()
    gather(j, rows0, sem0).wait()
    scatter(j, rows0)
    gather(j + 2, rows0, sem0).start()
    gather(j + 1, rows1, sem1).wait()
    scatter(j + 1, rows1)
    return 0

  lax.fori_loop(0, (NBLK - 1) // 2, pair, 0)
  gather(NBLK - 1, rows0, sem0).wait()
  scatter(NBLK - 1, rows0)

  plsc.subcore_barrier()
  pltpu.sync_copy(accum.at[pl.ds(sid * RPT, RPT)],
                  out_hbm.at[cid, pl.ds(sid * RPT, RPT)])


_spmm_call = pl.kernel(
    _spmm_body,
    out_type=jax.ShapeDtypeStruct((NC, N_PAD, D), jnp.float32),
    mesh=_mesh,
    scratch_types=[
        pltpu.VMEM((EPW,), jnp.int32),
        pltpu.VMEM((EPW,), jnp.int32),
        pltpu.VMEM((B, D), jnp.float32),
        pltpu.VMEM((B, D), jnp.float32),
        pltpu.VMEM((32, D), jnp.float32),
        pltpu.SemaphoreType.DMA,
        pltpu.SemaphoreType.DMA,
        pltpu.VMEM_SHARED((N_PAD, D), jnp.float32),
    ],
    compiler_params=_sc_params,
)


_RB = 1024
_GRID = N_PAD // _RB


def _lin1_body(x_ref, w_ref, d0_ref, d1_ref, y_ref, dinv_ref):
  dinv = lax.rsqrt(d0_ref[...] + d1_ref[...] + 1.0)
  h = jnp.dot(x_ref[...], w_ref[...], preferred_element_type=jnp.float32)
  y_ref[...] = dinv * h
  dinv_ref[...] = dinv


_lin1 = pl.pallas_call(
    _lin1_body,
    grid=(_GRID,),
    in_specs=[
        pl.BlockSpec((_RB, D), lambda i: (i, 0)),
        pl.BlockSpec((D, D), lambda i: (0, 0)),
        pl.BlockSpec((_RB, 1), lambda i: (i, 0)),
        pl.BlockSpec((_RB, 1), lambda i: (i, 0)),
    ],
    out_specs=[
        pl.BlockSpec((_RB, D), lambda i: (i, 0)),
        pl.BlockSpec((_RB, 1), lambda i: (i, 0)),
    ],
    out_shape=[
        jax.ShapeDtypeStruct((N_PAD, D), jnp.float32),
        jax.ShapeDtypeStruct((N_PAD, 1), jnp.float32),
    ],
)


def _lin2_body(z0_ref, z1_ref, y1_ref, dinv_ref, b1_ref, wc_ref, y2_ref):
  dinv = dinv_ref[...]
  h = dinv * (z0_ref[...] + z1_ref[...] + y1_ref[...]) + b1_ref[...]
  h = jnp.maximum(h, 0.0)
  y2_ref[...] = dinv * jnp.dot(h, wc_ref[...],
                               preferred_element_type=jnp.float32)


_lin2 = pl.pallas_call(
    _lin2_body,
    grid=(_GRID,),
    in_specs=[
        pl.BlockSpec((_RB, D), lambda i: (i, 0)),
        pl.BlockSpec((_RB, D), lambda i: (i, 0)),
        pl.BlockSpec((_RB, D), lambda i: (i, 0)),
        pl.BlockSpec((_RB, 1), lambda i: (i, 0)),
        pl.BlockSpec((1, D), lambda i: (0, 0)),
        pl.BlockSpec((D, D), lambda i: (0, 0)),
    ],
    out_specs=pl.BlockSpec((_RB, D), lambda i: (i, 0)),
    out_shape=jax.ShapeDtypeStruct((N_PAD, D), jnp.float32),
)


def _lin3_body(z0_ref, z1_ref, y2_ref, dinv_ref, bc_ref, o_ref):
  o_ref[...] = dinv_ref[...] * (z0_ref[...] + z1_ref[...] + y2_ref[...]) \
      + bc_ref[...]


_lin3 = pl.pallas_call(
    _lin3_body,
    grid=(_GRID,),
    in_specs=[
        pl.BlockSpec((_RB, D), lambda i: (i, 0)),
        pl.BlockSpec((_RB, D), lambda i: (i, 0)),
        pl.BlockSpec((_RB, D), lambda i: (i, 0)),
        pl.BlockSpec((_RB, 1), lambda i: (i, 0)),
        pl.BlockSpec((1, D), lambda i: (0, 0)),
    ],
    out_specs=pl.BlockSpec((_RB, D), lambda i: (i, 0)),
    out_shape=jax.ShapeDtypeStruct((N_PAD, D), jnp.float32),
)


def kernel(x, edge_index, W1, b1, W_mu, b_mu, W_logvar, b_logvar):
  xp = jnp.pad(x, ((0, N_PAD - N_NODES), (0, 0)))
  Wc = jnp.concatenate([W_mu, W_logvar], axis=1)
  bc = jnp.concatenate([b_mu, b_logvar]).reshape(1, D)

  src = edge_index[0]
  dst = edge_index[1]

  dg0, dg1 = _deg_call(dst)
  d0 = dg0.reshape(N_PAD, 1)
  d1 = dg1.reshape(N_PAD, 1)

  y1, dinv = _lin1(xp, W1, d0, d1)
  z1 = _spmm_call(y1, src, dst)
  y2 = _lin2(z1[0], z1[1], y1, dinv, b1.reshape(1, D), Wc)
  z2 = _spmm_call(y2, src, dst)
  o = _lin3(z2[0], z2[1], y2, dinv, bc)
  return o[:N_NODES, :LATENT], o[:N_NODES, LATENT:]


# speedup vs baseline: 34.5671x; 1.0082x over previous
import functools

import jax
import jax.numpy as jnp
from jax import lax
from jax.experimental import pallas as pl
from jax.experimental.pallas import tpu as pltpu
from jax.experimental.pallas import tpu_sc as plsc

N_NODES = 10000
N_PAD = 10240
N_EDGES = 320000
D = 128
LATENT = 64

NC = 2
NS = 16
NW = NC * NS
L = 16
EPW = N_EDGES // NW
B = 80
NBLK = EPW // B
DROWS = N_PAD // D
RPT = N_PAD // NS

_mesh = plsc.VectorSubcoreMesh(
    core_axis_name="c", subcore_axis_name="s", num_cores=NC, num_subcores=NS
)
_sc_params = pltpu.CompilerParams(needs_layout_passes=False)


def _zero_rows(ref, nrows):
  zero16 = jnp.zeros((L,), jnp.float32)

  def body(r, _):
    for c in range(D // L):
      ref[r, pl.ds(c * L, L)] = zero16
    return 0

  lax.fori_loop(0, nrows, body, 0)


_NPT = N_PAD // NS


def _deg_body(dst_hbm, out0_hbm, out1_hbm, dst_stage, deg_local, red, acc,
              slots):
  cid = lax.axis_index("c")
  sid = lax.axis_index("s")
  wid = sid * NC + cid

  zero16 = jnp.zeros((L,), jnp.float32)

  def zloc(i, _):
    deg_local[pl.ds(i * L, L)] = zero16
    return 0

  lax.fori_loop(0, N_PAD // L, zloc, 0)

  pltpu.sync_copy(dst_hbm.at[pl.ds(wid * EPW, EPW)], dst_stage)
  ones16 = jnp.ones((L,), jnp.float32)

  def hist(i, _):
    d = dst_stage[pl.ds(i * L, L)]
    plsc.addupdate_scatter(deg_local, [d], ones16)
    return 0

  lax.fori_loop(0, EPW // L, hist, 0)

  pltpu.sync_copy(deg_local, slots.at[sid])
  plsc.subcore_barrier()
  for k in range(NS):
    pltpu.sync_copy(slots.at[k, pl.ds(sid * _NPT, _NPT)], red.at[k])

  def rbody(j, _):
    s = red[0, pl.ds(j * L, L)]
    for k in range(1, NS):
      s = s + red[k, pl.ds(j * L, L)]
    acc[pl.ds(j * L, L)] = s
    return 0

  lax.fori_loop(0, _NPT // L, rbody, 0)

  @pl.when(cid == 0)
  def _():
    pltpu.sync_copy(acc, out0_hbm.at[pl.ds(sid * _NPT, _NPT)])

  @pl.when(cid == 1)
  def _():
    pltpu.sync_copy(acc, out1_hbm.at[pl.ds(sid * _NPT, _NPT)])


_deg_call = pl.kernel(
    _deg_body,
    out_type=(jax.ShapeDtypeStruct((N_PAD,), jnp.float32),
              jax.ShapeDtypeStruct((N_PAD,), jnp.float32)),
    mesh=_mesh,
    scratch_types=[
        pltpu.VMEM((EPW,), jnp.int32),
        pltpu.VMEM((N_PAD,), jnp.float32),
        pltpu.VMEM((NS, _NPT), jnp.float32),
        pltpu.VMEM((_NPT,), jnp.float32),
        pltpu.VMEM_SHARED((NS, N_PAD), jnp.float32),
    ],
    compiler_params=_sc_params,
)


def _spmm_body(y_hbm, src_hbm, dst_hbm, out_hbm, src_all, dst_all, rows0,
               rows1, zbuf, sem0, sem1, accum):
  cid = lax.axis_index("c")
  sid = lax.axis_index("s")
  wid = sid * NC + cid

  _zero_rows(zbuf, 32)
  for k in range(RPT // 32):
    pltpu.sync_copy(zbuf, accum.at[pl.ds(sid * RPT + k * 32, 32)])

  ebase = wid * EPW
  pltpu.sync_copy(src_hbm.at[pl.ds(ebase, EPW)], src_all)
  pltpu.sync_copy(dst_hbm.at[pl.ds(ebase, EPW)], dst_all)
  plsc.subcore_barrier()

  def gather(j, rows, sem):
    return pltpu.make_async_copy(
        y_hbm.at[src_all.at[pl.ds(j * B, B)]], rows, sem)

  def scatter(j, rows):
    pltpu.sync_copy(rows, accum.at[dst_all.at[pl.ds(j * B, B)]], add=True)

  gather(0, rows0, sem0).start()

  def pair(i, _):
    j = i * 2
    gather(j + 1, rows1, sem1).start()
    gather(j, rows0, sem0).wait()
    scatter(j, rows0)
    gather(j + 2, rows0, sem0).start()
    gather(j + 1, rows1, sem1).wait()
    scatter(j + 1, rows1)
    return 0

  lax.fori_loop(0, (NBLK - 1) // 2, pair, 0)
  gather(NBLK - 1, rows0, sem0).wait()
  scatter(NBLK - 1, rows0)

  plsc.subcore_barrier()
  pltpu.sync_copy(accum.at[pl.ds(sid * RPT, RPT)],
                  out_hbm.at[cid, pl.ds(sid * RPT, RPT)])


_spmm_call = pl.kernel(
    _spmm_body,
    out_type=jax.ShapeDtypeStruct((NC, N_PAD, D), jnp.float32),
    mesh=_mesh,
    scratch_types=[
        pltpu.VMEM((EPW,), jnp.int32),
        pltpu.VMEM((EPW,), jnp.int32),
        pltpu.VMEM((B, D), jnp.float32),
        pltpu.VMEM((B, D), jnp.float32),
        pltpu.VMEM((32, D), jnp.float32),
        pltpu.SemaphoreType.DMA,
        pltpu.SemaphoreType.DMA,
        pltpu.VMEM_SHARED((N_PAD, D), jnp.float32),
    ],
    compiler_params=_sc_params,
)


_RB = 1024
_GRID = N_PAD // _RB


def _lin1_body(x_ref, w_ref, d0_ref, d1_ref, y_ref, dinv_ref):
  dinv = lax.rsqrt(d0_ref[...] + d1_ref[...] + 1.0)
  h = jnp.dot(x_ref[...], w_ref[...], preferred_element_type=jnp.float32)
  y_ref[...] = dinv * h
  dinv_ref[...] = dinv


_lin1 = pl.pallas_call(
    _lin1_body,
    grid=(_GRID,),
    in_specs=[
        pl.BlockSpec((_RB, D), lambda i: (i, 0)),
        pl.BlockSpec((D, D), lambda i: (0, 0)),
        pl.BlockSpec((_RB, 1), lambda i: (i, 0)),
        pl.BlockSpec((_RB, 1), lambda i: (i, 0)),
    ],
    out_specs=[
        pl.BlockSpec((_RB, D), lambda i: (i, 0)),
        pl.BlockSpec((_RB, 1), lambda i: (i, 0)),
    ],
    out_shape=[
        jax.ShapeDtypeStruct((N_PAD, D), jnp.float32),
        jax.ShapeDtypeStruct((N_PAD, 1), jnp.float32),
    ],
)


def _lin2_body(z0_ref, z1_ref, y1_ref, dinv_ref, b1_ref, wmu_ref, wlv_ref,
               y2_ref):
  dinv = dinv_ref[...]
  h = dinv * (z0_ref[...] + z1_ref[...] + y1_ref[...]) + b1_ref[...]
  h = jnp.maximum(h, 0.0)
  y2_ref[:, :LATENT] = dinv * jnp.dot(h, wmu_ref[...],
                                      preferred_element_type=jnp.float32)
  y2_ref[:, LATENT:] = dinv * jnp.dot(h, wlv_ref[...],
                                      preferred_element_type=jnp.float32)


_lin2 = pl.pallas_call(
    _lin2_body,
    grid=(_GRID,),
    in_specs=[
        pl.BlockSpec((_RB, D), lambda i: (i, 0)),
        pl.BlockSpec((_RB, D), lambda i: (i, 0)),
        pl.BlockSpec((_RB, D), lambda i: (i, 0)),
        pl.BlockSpec((_RB, 1), lambda i: (i, 0)),
        pl.BlockSpec((1, D), lambda i: (0, 0)),
        pl.BlockSpec((D, LATENT), lambda i: (0, 0)),
        pl.BlockSpec((D, LATENT), lambda i: (0, 0)),
    ],
    out_specs=pl.BlockSpec((_RB, D), lambda i: (i, 0)),
    out_shape=jax.ShapeDtypeStruct((N_PAD, D), jnp.float32),
)


def _lin3_body(z0_ref, z1_ref, y2_ref, dinv_ref, bmu_ref, blv_ref, mu_ref,
               lv_ref):
  o = dinv_ref[...] * (z0_ref[...] + z1_ref[...] + y2_ref[...])
  mu_ref[...] = o[:, :LATENT] + bmu_ref[...]
  lv_ref[...] = o[:, LATENT:] + blv_ref[...]


_lin3 = pl.pallas_call(
    _lin3_body,
    grid=(_GRID,),
    in_specs=[
        pl.BlockSpec((_RB, D), lambda i: (i, 0)),
        pl.BlockSpec((_RB, D), lambda i: (i, 0)),
        pl.BlockSpec((_RB, D), lambda i: (i, 0)),
        pl.BlockSpec((_RB, 1), lambda i: (i, 0)),
        pl.BlockSpec((1, LATENT), lambda i: (0, 0)),
        pl.BlockSpec((1, LATENT), lambda i: (0, 0)),
    ],
    out_specs=[
        pl.BlockSpec((_RB, LATENT), lambda i: (i, 0)),
        pl.BlockSpec((_RB, LATENT), lambda i: (i, 0)),
    ],
    out_shape=[
        jax.ShapeDtypeStruct((N_NODES, LATENT), jnp.float32),
        jax.ShapeDtypeStruct((N_NODES, LATENT), jnp.float32),
    ],
)


def kernel(x, edge_index, W1, b1, W_mu, b_mu, W_logvar, b_logvar):
  xp = jnp.pad(x, ((0, N_PAD - N_NODES), (0, 0)))
  src = edge_index[0]
  dst = edge_index[1]

  dg0, dg1 = _deg_call(dst)
  d0 = dg0.reshape(N_PAD, 1)
  d1 = dg1.reshape(N_PAD, 1)

  y1, dinv = _lin1(xp, W1, d0, d1)
  z1 = _spmm_call(y1, src, dst)
  y2 = _lin2(z1[0], z1[1], y1, dinv, b1.reshape(1, D), W_mu, W_logvar)
  z2 = _spmm_call(y2, src, dst)
  mu, lv = _lin3(z2[0], z2[1], y2, dinv, b_mu.reshape(1, LATENT),
                 b_logvar.reshape(1, LATENT))
  return mu, lv


# speedup vs baseline: 35.1801x; 1.0177x over previous
import functools

import jax
import jax.numpy as jnp
from jax import lax
from jax.experimental import pallas as pl
from jax.experimental.pallas import tpu as pltpu
from jax.experimental.pallas import tpu_sc as plsc

N_NODES = 10000
N_PAD = 10240
N_EDGES = 320000
D = 128
LATENT = 64

NC = 2
NS = 16
NW = NC * NS
L = 16
EPW = N_EDGES // NW
B = 80
NBLK = EPW // B
DROWS = N_PAD // D
RPT = N_PAD // NS

_mesh = plsc.VectorSubcoreMesh(
    core_axis_name="c", subcore_axis_name="s", num_cores=NC, num_subcores=NS
)
_sc_params = pltpu.CompilerParams(needs_layout_passes=False)


def _zero_rows(ref, nrows):
  zero16 = jnp.zeros((L,), jnp.float32)

  def body(r, _):
    for c in range(D // L):
      ref[r, pl.ds(c * L, L)] = zero16
    return 0

  lax.fori_loop(0, nrows, body, 0)


_NPT = N_PAD // NS


def _deg_body(dst_hbm, out0_hbm, out1_hbm, dst_stage, deg_local, red, acc,
              slots):
  cid = lax.axis_index("c")
  sid = lax.axis_index("s")
  wid = sid * NC + cid

  zero16 = jnp.zeros((L,), jnp.float32)

  def zloc(i, _):
    deg_local[pl.ds(i * L, L)] = zero16
    return 0

  lax.fori_loop(0, N_PAD // L, zloc, 0)

  pltpu.sync_copy(dst_hbm.at[pl.ds(wid * EPW, EPW)], dst_stage)
  ones16 = jnp.ones((L,), jnp.float32)

  def hist(i, _):
    d = dst_stage[pl.ds(i * L, L)]
    plsc.addupdate_scatter(deg_local, [d], ones16)
    return 0

  lax.fori_loop(0, EPW // L, hist, 0)

  pltpu.sync_copy(deg_local, slots.at[sid])
  plsc.subcore_barrier()
  for k in range(NS):
    pltpu.sync_copy(slots.at[k, pl.ds(sid * _NPT, _NPT)], red.at[k])

  def rbody(j, _):
    s = red[0, pl.ds(j * L, L)]
    for k in range(1, NS):
      s = s + red[k, pl.ds(j * L, L)]
    acc[pl.ds(j * L, L)] = s
    return 0

  lax.fori_loop(0, _NPT // L, rbody, 0)

  @pl.when(cid == 0)
  def _():
    pltpu.sync_copy(acc, out0_hbm.at[pl.ds(sid * _NPT, _NPT)])

  @pl.when(cid == 1)
  def _():
    pltpu.sync_copy(acc, out1_hbm.at[pl.ds(sid * _NPT, _NPT)])


_deg_call = pl.kernel(
    _deg_body,
    out_type=(jax.ShapeDtypeStruct((N_PAD,), jnp.float32),
              jax.ShapeDtypeStruct((N_PAD,), jnp.float32)),
    mesh=_mesh,
    scratch_types=[
        pltpu.VMEM((EPW,), jnp.int32),
        pltpu.VMEM((N_PAD,), jnp.float32),
        pltpu.VMEM((NS, _NPT), jnp.float32),
        pltpu.VMEM((_NPT,), jnp.float32),
        pltpu.VMEM_SHARED((NS, N_PAD), jnp.float32),
    ],
    compiler_params=_sc_params,
)


def _spmm_body(y_hbm, src_hbm, dst_hbm, out_hbm, src_all, dst_all, rows0,
               rows1, zbuf, sem0, sem1, semz, accum):
  cid = lax.axis_index("c")
  sid = lax.axis_index("s")
  wid = sid * NC + cid

  ebase = wid * EPW
  stage_s = pltpu.make_async_copy(src_hbm.at[pl.ds(ebase, EPW)], src_all,
                                  sem0)
  stage_d = pltpu.make_async_copy(dst_hbm.at[pl.ds(ebase, EPW)], dst_all,
                                  sem1)
  stage_s.start()
  stage_d.start()

  _zero_rows(zbuf, 64)
  zcopies = [
      pltpu.make_async_copy(zbuf, accum.at[pl.ds(sid * RPT + k * 64, 64)],
                            semz) for k in range(RPT // 64)
  ]
  for zc in zcopies:
    zc.start()
  for zc in zcopies:
    zc.wait()
  stage_s.wait()
  stage_d.wait()
  plsc.subcore_barrier()

  def gather(j, rows, sem):
    return pltpu.make_async_copy(
        y_hbm.at[src_all.at[pl.ds(j * B, B)]], rows, sem)

  def scatter(j, rows):
    pltpu.sync_copy(rows, accum.at[dst_all.at[pl.ds(j * B, B)]], add=True)

  gather(0, rows0, sem0).start()

  def pair(i, _):
    j = i * 2
    gather(j + 1, rows1, sem1).start()
    gather(j, rows0, sem0).wait()
    scatter(j, rows0)
    gather(j + 2, rows0, sem0).start()
    gather(j + 1, rows1, sem1).wait()
    scatter(j + 1, rows1)
    return 0

  lax.fori_loop(0, (NBLK - 1) // 2, pair, 0)
  gather(NBLK - 1, rows0, sem0).wait()
  scatter(NBLK - 1, rows0)

  plsc.subcore_barrier()
  pltpu.sync_copy(accum.at[pl.ds(sid * RPT, RPT)],
                  out_hbm.at[cid, pl.ds(sid * RPT, RPT)])


_spmm_call = pl.kernel(
    _spmm_body,
    out_type=jax.ShapeDtypeStruct((NC, N_PAD, D), jnp.float32),
    mesh=_mesh,
    scratch_types=[
        pltpu.VMEM((EPW,), jnp.int32),
        pltpu.VMEM((EPW,), jnp.int32),
        pltpu.VMEM((B, D), jnp.float32),
        pltpu.VMEM((B, D), jnp.float32),
        pltpu.VMEM((64, D), jnp.float32),
        pltpu.SemaphoreType.DMA,
        pltpu.SemaphoreType.DMA,
        pltpu.SemaphoreType.DMA,
        pltpu.VMEM_SHARED((N_PAD, D), jnp.float32),
    ],
    compiler_params=_sc_params,
)


_RB = 1024
_GRID = N_PAD // _RB


def _lin1_body(x_ref, w_ref, d0_ref, d1_ref, y_ref, dinv_ref):
  dinv = lax.rsqrt(d0_ref[...] + d1_ref[...] + 1.0)
  h = jnp.dot(x_ref[...], w_ref[...], preferred_element_type=jnp.float32)
  y_ref[...] = dinv * h
  dinv_ref[...] = dinv


_lin1 = pl.pallas_call(
    _lin1_body,
    grid=(_GRID,),
    in_specs=[
        pl.BlockSpec((_RB, D), lambda i: (i, 0)),
        pl.BlockSpec((D, D), lambda i: (0, 0)),
        pl.BlockSpec((_RB, 1), lambda i: (i, 0)),
        pl.BlockSpec((_RB, 1), lambda i: (i, 0)),
    ],
    out_specs=[
        pl.BlockSpec((_RB, D), lambda i: (i, 0)),
        pl.BlockSpec((_RB, 1), lambda i: (i, 0)),
    ],
    out_shape=[
        jax.ShapeDtypeStruct((N_PAD, D), jnp.float32),
        jax.ShapeDtypeStruct((N_PAD, 1), jnp.float32),
    ],
)


def _lin2_body(z0_ref, z1_ref, y1_ref, dinv_ref, b1_ref, wmu_ref, wlv_ref,
               y2_ref):
  dinv = dinv_ref[...]
  h = dinv * (z0_ref[...] + z1_ref[...] + y1_ref[...]) + b1_ref[...]
  h = jnp.maximum(h, 0.0)
  y2_ref[:, :LATENT] = dinv * jnp.dot(h, wmu_ref[...],
                                      preferred_element_type=jnp.float32)
  y2_ref[:, LATENT:] = dinv * jnp.dot(h, wlv_ref[...],
                                      preferred_element_type=jnp.float32)


_lin2 = pl.pallas_call(
    _lin2_body,
    grid=(_GRID,),
    in_specs=[
        pl.BlockSpec((_RB, D), lambda i: (i, 0)),
        pl.BlockSpec((_RB, D), lambda i: (i, 0)),
        pl.BlockSpec((_RB, D), lambda i: (i, 0)),
        pl.BlockSpec((_RB, 1), lambda i: (i, 0)),
        pl.BlockSpec((1, D), lambda i: (0, 0)),
        pl.BlockSpec((D, LATENT), lambda i: (0, 0)),
        pl.BlockSpec((D, LATENT), lambda i: (0, 0)),
    ],
    out_specs=pl.BlockSpec((_RB, D), lambda i: (i, 0)),
    out_shape=jax.ShapeDtypeStruct((N_PAD, D), jnp.float32),
)


def _lin3_body(z0_ref, z1_ref, y2_ref, dinv_ref, bmu_ref, blv_ref, mu_ref,
               lv_ref):
  o = dinv_ref[...] * (z0_ref[...] + z1_ref[...] + y2_ref[...])
  mu_ref[...] = o[:, :LATENT] + bmu_ref[...]
  lv_ref[...] = o[:, LATENT:] + blv_ref[...]


_lin3 = pl.pallas_call(
    _lin3_body,
    grid=(_GRID,),
    in_specs=[
        pl.BlockSpec((_RB, D), lambda i: (i, 0)),
        pl.BlockSpec((_RB, D), lambda i: (i, 0)),
        pl.BlockSpec((_RB, D), lambda i: (i, 0)),
        pl.BlockSpec((_RB, 1), lambda i: (i, 0)),
        pl.BlockSpec((1, LATENT), lambda i: (0, 0)),
        pl.BlockSpec((1, LATENT), lambda i: (0, 0)),
    ],
    out_specs=[
        pl.BlockSpec((_RB, LATENT), lambda i: (i, 0)),
        pl.BlockSpec((_RB, LATENT), lambda i: (i, 0)),
    ],
    out_shape=[
        jax.ShapeDtypeStruct((N_NODES, LATENT), jnp.float32),
        jax.ShapeDtypeStruct((N_NODES, LATENT), jnp.float32),
    ],
)


def kernel(x, edge_index, W1, b1, W_mu, b_mu, W_logvar, b_logvar):
  xp = jnp.pad(x, ((0, N_PAD - N_NODES), (0, 0)))
  src = edge_index[0]
  dst = edge_index[1]

  dg0, dg1 = _deg_call(dst)
  d0 = dg0.reshape(N_PAD, 1)
  d1 = dg1.reshape(N_PAD, 1)

  y1, dinv = _lin1(xp, W1, d0, d1)
  z1 = _spmm_call(y1, src, dst)
  y2 = _lin2(z1[0], z1[1], y1, dinv, b1.reshape(1, D), W_mu, W_logvar)
  z2 = _spmm_call(y2, src, dst)
  mu, lv = _lin3(z2[0], z2[1], y2, dinv, b_mu.reshape(1, LATENT),
                 b_logvar.reshape(1, LATENT))
  return mu, lv


# speedup vs baseline: 38.1574x; 1.0846x over previous
import functools

import jax
import jax.numpy as jnp
from jax import lax
from jax.experimental import pallas as pl
from jax.experimental.pallas import tpu as pltpu
from jax.experimental.pallas import tpu_sc as plsc

N_NODES = 10000
N_PAD = 10240
N_EDGES = 320000
D = 128
LATENT = 64

NC = 2
NS = 16
NW = NC * NS
L = 16
EPW = N_EDGES // NW
B = 80
NBLK = EPW // B
DROWS = N_PAD // D
RPT = N_PAD // NS

_mesh = plsc.VectorSubcoreMesh(
    core_axis_name="c", subcore_axis_name="s", num_cores=NC, num_subcores=NS
)
_sc_params = pltpu.CompilerParams(needs_layout_passes=False)


def _zero_rows(ref, nrows):
  zero16 = jnp.zeros((L,), jnp.float32)

  def body(r, _):
    for c in range(D // L):
      ref[r, pl.ds(c * L, L)] = zero16
    return 0

  lax.fori_loop(0, nrows, body, 0)


_NPT = N_PAD // NS


def _deg_body(eflat_hbm, out0_hbm, out1_hbm, dst_stage, deg_local, red, acc,
              slots):
  cid = lax.axis_index("c")
  sid = lax.axis_index("s")
  wid = sid * NC + cid

  zero16 = jnp.zeros((L,), jnp.float32)

  def zloc(i, _):
    deg_local[pl.ds(i * L, L)] = zero16
    return 0

  lax.fori_loop(0, N_PAD // L, zloc, 0)

  pltpu.sync_copy(eflat_hbm.at[pl.ds(N_EDGES + wid * EPW, EPW)], dst_stage)
  ones16 = jnp.ones((L,), jnp.float32)

  def hist(i, _):
    d = dst_stage[pl.ds(i * L, L)]
    plsc.addupdate_scatter(deg_local, [d], ones16)
    return 0

  lax.fori_loop(0, EPW // L, hist, 0)

  pltpu.sync_copy(deg_local, slots.at[sid])
  plsc.subcore_barrier()
  for k in range(NS):
    pltpu.sync_copy(slots.at[k, pl.ds(sid * _NPT, _NPT)], red.at[k])

  def rbody(j, _):
    s = red[0, pl.ds(j * L, L)]
    for k in range(1, NS):
      s = s + red[k, pl.ds(j * L, L)]
    acc[pl.ds(j * L, L)] = s
    return 0

  lax.fori_loop(0, _NPT // L, rbody, 0)

  @pl.when(cid == 0)
  def _():
    pltpu.sync_copy(acc, out0_hbm.at[pl.ds(sid * _NPT, _NPT)])

  @pl.when(cid == 1)
  def _():
    pltpu.sync_copy(acc, out1_hbm.at[pl.ds(sid * _NPT, _NPT)])


_deg_call = pl.kernel(
    _deg_body,
    out_type=(jax.ShapeDtypeStruct((N_PAD,), jnp.float32),
              jax.ShapeDtypeStruct((N_PAD,), jnp.float32)),
    mesh=_mesh,
    scratch_types=[
        pltpu.VMEM((EPW,), jnp.int32),
        pltpu.VMEM((N_PAD,), jnp.float32),
        pltpu.VMEM((NS, _NPT), jnp.float32),
        pltpu.VMEM((_NPT,), jnp.float32),
        pltpu.VMEM_SHARED((NS, N_PAD), jnp.float32),
    ],
    compiler_params=_sc_params,
)


def _spmm_body(y_hbm, eflat_hbm, out_hbm, src_all, dst_all, rows0,
               rows1, zbuf, sem0, sem1, semz, accum):
  cid = lax.axis_index("c")
  sid = lax.axis_index("s")
  wid = sid * NC + cid

  ebase = wid * EPW
  stage_s = pltpu.make_async_copy(eflat_hbm.at[pl.ds(ebase, EPW)], src_all,
                                  sem0)
  stage_d = pltpu.make_async_copy(eflat_hbm.at[pl.ds(N_EDGES + ebase, EPW)],
                                  dst_all, sem1)
  stage_s.start()
  stage_d.start()

  _zero_rows(zbuf, 64)
  zcopies = [
      pltpu.make_async_copy(zbuf, accum.at[pl.ds(sid * RPT + k * 64, 64)],
                            semz) for k in range(RPT // 64)
  ]
  for zc in zcopies:
    zc.start()
  for zc in zcopies:
    zc.wait()
  stage_s.wait()
  stage_d.wait()
  plsc.subcore_barrier()

  def gather(j, rows, sem):
    return pltpu.make_async_copy(
        y_hbm.at[src_all.at[pl.ds(j * B, B)]], rows, sem)

  def scatter(j, rows):
    pltpu.sync_copy(rows, accum.at[dst_all.at[pl.ds(j * B, B)]], add=True)

  gather(0, rows0, sem0).start()

  def pair(i, _):
    j = i * 2
    gather(j + 1, rows1, sem1).start()
    gather(j, rows0, sem0).wait()
    scatter(j, rows0)
    gather(j + 2, rows0, sem0).start()
    gather(j + 1, rows1, sem1).wait()
    scatter(j + 1, rows1)
    return 0

  lax.fori_loop(0, (NBLK - 1) // 2, pair, 0)
  gather(NBLK - 1, rows0, sem0).wait()
  scatter(NBLK - 1, rows0)

  plsc.subcore_barrier()
  pltpu.sync_copy(accum.at[pl.ds(sid * RPT, RPT)],
                  out_hbm.at[cid, pl.ds(sid * RPT, RPT)])


_spmm_call = pl.kernel(
    _spmm_body,
    out_type=jax.ShapeDtypeStruct((NC, N_PAD, D), jnp.float32),
    mesh=_mesh,
    scratch_types=[
        pltpu.VMEM((EPW,), jnp.int32),
        pltpu.VMEM((EPW,), jnp.int32),
        pltpu.VMEM((B, D), jnp.float32),
        pltpu.VMEM((B, D), jnp.float32),
        pltpu.VMEM((64, D), jnp.float32),
        pltpu.SemaphoreType.DMA,
        pltpu.SemaphoreType.DMA,
        pltpu.SemaphoreType.DMA,
        pltpu.VMEM_SHARED((N_PAD, D), jnp.float32),
    ],
    compiler_params=_sc_params,
)


_RB = 1024
_GRID = N_PAD // _RB


def _lin1_body(x_ref, w_ref, d0_ref, d1_ref, y_ref, dinv_ref):
  dinv = lax.rsqrt(d0_ref[...] + d1_ref[...] + 1.0)
  h = jnp.dot(x_ref[...], w_ref[...], preferred_element_type=jnp.float32)
  y_ref[...] = dinv * h
  dinv_ref[...] = dinv


_lin1 = pl.pallas_call(
    _lin1_body,
    grid=(_GRID,),
    in_specs=[
        pl.BlockSpec((_RB, D), lambda i: (i, 0)),
        pl.BlockSpec((D, D), lambda i: (0, 0)),
        pl.BlockSpec((_RB, 1), lambda i: (i, 0)),
        pl.BlockSpec((_RB, 1), lambda i: (i, 0)),
    ],
    out_specs=[
        pl.BlockSpec((_RB, D), lambda i: (i, 0)),
        pl.BlockSpec((_RB, 1), lambda i: (i, 0)),
    ],
    out_shape=[
        jax.ShapeDtypeStruct((N_PAD, D), jnp.float32),
        jax.ShapeDtypeStruct((N_PAD, 1), jnp.float32),
    ],
)


def _lin2_body(z_ref, y1_ref, dinv_ref, b1_ref, wmu_ref, wlv_ref,
               y2_ref):
  dinv = dinv_ref[...]
  h = dinv * (z_ref[0] + z_ref[1] + y1_ref[...]) + b1_ref[...]
  h = jnp.maximum(h, 0.0)
  y2_ref[:, :LATENT] = dinv * jnp.dot(h, wmu_ref[...],
                                      preferred_element_type=jnp.float32)
  y2_ref[:, LATENT:] = dinv * jnp.dot(h, wlv_ref[...],
                                      preferred_element_type=jnp.float32)


_lin2 = pl.pallas_call(
    _lin2_body,
    grid=(_GRID,),
    in_specs=[
        pl.BlockSpec((NC, _RB, D), lambda i: (0, i, 0)),
        pl.BlockSpec((_RB, D), lambda i: (i, 0)),
        pl.BlockSpec((_RB, 1), lambda i: (i, 0)),
        pl.BlockSpec((1, D), lambda i: (0, 0)),
        pl.BlockSpec((D, LATENT), lambda i: (0, 0)),
        pl.BlockSpec((D, LATENT), lambda i: (0, 0)),
    ],
    out_specs=pl.BlockSpec((_RB, D), lambda i: (i, 0)),
    out_shape=jax.ShapeDtypeStruct((N_PAD, D), jnp.float32),
)


def _lin3_body(z_ref, y2_ref, dinv_ref, bmu_ref, blv_ref, mu_ref,
               lv_ref):
  o = dinv_ref[...] * (z_ref[0] + z_ref[1] + y2_ref[...])
  mu_ref[...] = o[:, :LATENT] + bmu_ref[...]
  lv_ref[...] = o[:, LATENT:] + blv_ref[...]


_lin3 = pl.pallas_call(
    _lin3_body,
    grid=(_GRID,),
    in_specs=[
        pl.BlockSpec((NC, _RB, D), lambda i: (0, i, 0)),
        pl.BlockSpec((_RB, D), lambda i: (i, 0)),
        pl.BlockSpec((_RB, 1), lambda i: (i, 0)),
        pl.BlockSpec((1, LATENT), lambda i: (0, 0)),
        pl.BlockSpec((1, LATENT), lambda i: (0, 0)),
    ],
    out_specs=[
        pl.BlockSpec((_RB, LATENT), lambda i: (i, 0)),
        pl.BlockSpec((_RB, LATENT), lambda i: (i, 0)),
    ],
    out_shape=[
        jax.ShapeDtypeStruct((N_NODES, LATENT), jnp.float32),
        jax.ShapeDtypeStruct((N_NODES, LATENT), jnp.float32),
    ],
)


def kernel(x, edge_index, W1, b1, W_mu, b_mu, W_logvar, b_logvar):
  eflat = edge_index.reshape(2 * N_EDGES)

  dg0, dg1 = _deg_call(eflat)
  d0 = dg0.reshape(N_PAD, 1)
  d1 = dg1.reshape(N_PAD, 1)

  y1, dinv = _lin1(x, W1, d0, d1)
  z1 = _spmm_call(y1, eflat)
  y2 = _lin2(z1, y1, dinv, b1.reshape(1, D), W_mu, W_logvar)
  z2 = _spmm_call(y2, eflat)
  mu, lv = _lin3(z2, y2, dinv, b_mu.reshape(1, LATENT),
                 b_logvar.reshape(1, LATENT))
  return mu, lv
